# baseline (speedup 1.0000x reference)
import functools

import jax
import jax.numpy as jnp
from jax import lax
from jax.experimental import pallas as pl
from jax.experimental.pallas import tpu as pltpu
from jax.experimental.pallas import tpu_sc as plsc

NUM_ITEMS = 1000000
LATENT_DIM = 64
BATCH = 16384

NC = 2
NS = 16
L = 16
NW = NC * NS
BPW = BATCH // NW
CHUNK = 128
NCHUNK = BPW // CHUNK
PAIRW = 2 * LATENT_DIM


def _body(table_hbm, idx_hbm, pidx_hbm, w_hbm, b_hbm, out_hbm,
          idx_v, par_v, rows_v, out_v, w_v, b_v, tr_v, sem):
    wid = lax.axis_index("s") * NC + lax.axis_index("c")
    base = wid * BPW

    pltpu.sync_copy(w_hbm, w_v)
    pltpu.sync_copy(b_hbm, b_v)
    for c in range(NCHUNK):
        pltpu.sync_copy(pidx_hbm.at[pl.ds(base + c * CHUNK, CHUNK)],
                        idx_v.at[c])
    pltpu.sync_copy(idx_hbm.at[pl.ds(base, BPW)], par_v)
    copies = []
    for c in range(NCHUNK):
        copies.append(pltpu.async_copy(
            table_hbm.at[idx_v.at[c]],
            rows_v.at[pl.ds(c * CHUNK, CHUNK)], sem))
    for cp in copies:
        cp.wait()

    w0 = w_v[pl.ds(0, L)]
    w1 = w_v[pl.ds(L, L)]
    w2 = w_v[pl.ds(2 * L, L)]
    w3 = w_v[pl.ds(3 * L, L)]
    bv = b_v[...]
    colbase = lax.iota(jnp.int32, L) * L
    one = jnp.ones((L,), jnp.int32)

    def group(g, carry):
        for i in range(L):
            r = g * L + i
            pa = bv + rows_v[r, pl.ds(0, L)] * w0
            pa = pa + rows_v[r, pl.ds(L, L)] * w1
            pa = pa + rows_v[r, pl.ds(2 * L, L)] * w2
            pa = pa + rows_v[r, pl.ds(3 * L, L)] * w3
            tr_v[pl.ds(i * L, L)] = pa
            pb = bv + rows_v[r, pl.ds(4 * L, L)] * w0
            pb = pb + rows_v[r, pl.ds(5 * L, L)] * w1
            pb = pb + rows_v[r, pl.ds(6 * L, L)] * w2
            pb = pb + rows_v[r, pl.ds(7 * L, L)] * w3
            tr_v[pl.ds((L + i) * L, L)] = pb
        sa = plsc.load_gather(tr_v, [colbase])
        sb = plsc.load_gather(tr_v, [colbase + L * L])
        for l in range(1, L):
            sa = sa + plsc.load_gather(tr_v, [colbase + l])
            sb = sb + plsc.load_gather(tr_v, [colbase + L * L + l])
        par = par_v[pl.ds(g * L, L)] & one
        s = jnp.where(par == one, sb, sa)
        out_v[pl.ds(g * L, L)] = s
        return carry

    lax.fori_loop(0, BPW // L, group, 0)

    pltpu.sync_copy(out_v, out_hbm.at[pl.ds(base, BPW)])


@jax.jit
def kernel(item_indices, item_emb, ffn_w, ffn_b):
    idx = item_indices.astype(jnp.int32)
    pidx = lax.shift_right_logical(idx, 1)
    table = item_emb.reshape(NUM_ITEMS // 2, PAIRW)
    w = ffn_w.reshape(LATENT_DIM).astype(jnp.float32)
    bvec = jnp.pad(ffn_b.astype(jnp.float32), (0, L - 1))

    run = pl.kernel(
        _body,
        out_type=jax.ShapeDtypeStruct((BATCH,), jnp.float32),
        mesh=plsc.VectorSubcoreMesh(core_axis_name="c", subcore_axis_name="s",
                                    num_cores=NC, num_subcores=NS),
        compiler_params=pltpu.CompilerParams(needs_layout_passes=False),
        scratch_types=[
            pltpu.VMEM((NCHUNK, CHUNK), jnp.int32),
            pltpu.VMEM((BPW,), jnp.int32),
            pltpu.VMEM((BPW, PAIRW), jnp.float32),
            pltpu.VMEM((BPW,), jnp.float32),
            pltpu.VMEM((LATENT_DIM,), jnp.float32),
            pltpu.VMEM((L,), jnp.float32),
            pltpu.VMEM((2 * L * L,), jnp.float32),
            pltpu.SemaphoreType.DMA,
        ],
    )
    out = run(table, idx, pidx, w, bvec)
    return out.reshape(BATCH, 1)


# speedup vs baseline: 3.3853x; 3.3853x over previous
import functools

import jax
import jax.numpy as jnp
from jax import lax
from jax.experimental import pallas as pl
from jax.experimental.pallas import tpu as pltpu
from jax.experimental.pallas import tpu_sc as plsc

NUM_ITEMS = 1000000
LATENT_DIM = 64
BATCH = 16384

NC = 2
NS = 16
NW = NC * NS
BPW = BATCH // NW
CHUNK = 128
NCHUNK = BPW // CHUNK

BLKW = 8192
NBLK = (NUM_ITEMS + BLKW - 1) // BLKW


def _matvec_body(w_ref, et_ref, b_ref, y_ref):
    y_ref[...] = jnp.dot(w_ref[...], et_ref[...],
                         preferred_element_type=jnp.float32) + b_ref[0, 0]


def _gather_body(y_hbm, idx_hbm, out_hbm, idx_v, val_v, sem):
    wid = lax.axis_index("s") * NC + lax.axis_index("c")
    base = wid * BPW
    for c in range(NCHUNK):
        pltpu.sync_copy(idx_hbm.at[pl.ds(base + c * CHUNK, CHUNK)],
                        idx_v.at[c])
    copies = []
    for c in range(NCHUNK):
        copies.append(pltpu.async_copy(
            y_hbm.at[idx_v.at[c]],
            val_v.at[pl.ds(c * CHUNK, CHUNK)], sem))
    for cp in copies:
        cp.wait()
    pltpu.sync_copy(val_v, out_hbm.at[pl.ds(base, BPW)])


@jax.jit
def kernel(item_indices, item_emb, ffn_w, ffn_b):
    idx = item_indices.astype(jnp.int32)
    et = jnp.swapaxes(item_emb, 0, 1)
    b2 = ffn_b.reshape(1, 1)

    y2 = pl.pallas_call(
        _matvec_body,
        grid=(NBLK,),
        in_specs=[
            pl.BlockSpec((1, LATENT_DIM), lambda i: (0, 0)),
            pl.BlockSpec((LATENT_DIM, BLKW), lambda i: (0, i)),
            pl.BlockSpec((1, 1), lambda i: (0, 0), memory_space=pltpu.SMEM),
        ],
        out_specs=pl.BlockSpec((1, BLKW), lambda i: (0, i)),
        out_shape=jax.ShapeDtypeStruct((1, NUM_ITEMS), jnp.float32),
    )(ffn_w, et, b2)
    y = y2.reshape(NUM_ITEMS)

    run = pl.kernel(
        _gather_body,
        out_type=jax.ShapeDtypeStruct((BATCH,), jnp.float32),
        mesh=plsc.VectorSubcoreMesh(core_axis_name="c", subcore_axis_name="s",
                                    num_cores=NC, num_subcores=NS),
        compiler_params=pltpu.CompilerParams(needs_layout_passes=False),
        scratch_types=[
            pltpu.VMEM((NCHUNK, CHUNK), jnp.int32),
            pltpu.VMEM((BPW,), jnp.float32),
            pltpu.SemaphoreType.DMA,
        ],
    )
    out = run(y, idx)
    return out.reshape(BATCH, 1)


# speedup vs baseline: 4.4965x; 1.3282x over previous
import functools

import jax
import jax.numpy as jnp
from jax import lax
from jax.experimental import pallas as pl
from jax.experimental.pallas import tpu as pltpu
from jax.experimental.pallas import tpu_sc as plsc

NUM_ITEMS = 1000000
LATENT_DIM = 64
BATCH = 16384

NC = 2
NS = 16
NW = NC * NS
BPW = BATCH // NW
CHUNK = 128
NCHUNK = BPW // CHUNK

BLKW = 32768
NBLK = (NUM_ITEMS + BLKW - 1) // BLKW


def _matvec_body(w_ref, et_ref, b_ref, y_ref):
    y_ref[...] = jnp.dot(w_ref[...], et_ref[...],
                         preferred_element_type=jnp.float32) + b_ref[0, 0]


def _gather_body(y_hbm, idx_hbm, out_hbm, idx_v, val_v, sem):
    wid = lax.axis_index("s") * NC + lax.axis_index("c")
    base = wid * BPW
    for c in range(NCHUNK):
        pltpu.sync_copy(idx_hbm.at[pl.ds(base + c * CHUNK, CHUNK)],
                        idx_v.at[c])
    copies = []
    for c in range(NCHUNK):
        copies.append(pltpu.async_copy(
            y_hbm.at[idx_v.at[c]],
            val_v.at[pl.ds(c * CHUNK, CHUNK)], sem))
    for cp in copies:
        cp.wait()
    pltpu.sync_copy(val_v, out_hbm.at[pl.ds(base, BPW)])


@jax.jit
def kernel(item_indices, item_emb, ffn_w, ffn_b):
    idx = item_indices.astype(jnp.int32)
    et = jnp.swapaxes(item_emb, 0, 1)
    b2 = ffn_b.reshape(1, 1)

    y2 = pl.pallas_call(
        _matvec_body,
        grid=(NBLK,),
        in_specs=[
            pl.BlockSpec((1, LATENT_DIM), lambda i: (0, 0)),
            pl.BlockSpec((LATENT_DIM, BLKW), lambda i: (0, i)),
            pl.BlockSpec((1, 1), lambda i: (0, 0), memory_space=pltpu.SMEM),
        ],
        out_specs=pl.BlockSpec((1, BLKW), lambda i: (0, i)),
        out_shape=jax.ShapeDtypeStruct((1, NUM_ITEMS), jnp.float32),
    )(ffn_w, et, b2)
    y = y2.reshape(NUM_ITEMS)

    run = pl.kernel(
        _gather_body,
        out_type=jax.ShapeDtypeStruct((BATCH,), jnp.float32),
        mesh=plsc.VectorSubcoreMesh(core_axis_name="c", subcore_axis_name="s",
                                    num_cores=NC, num_subcores=NS),
        compiler_params=pltpu.CompilerParams(needs_layout_passes=False),
        scratch_types=[
            pltpu.VMEM((NCHUNK, CHUNK), jnp.int32),
            pltpu.VMEM((BPW,), jnp.float32),
            pltpu.SemaphoreType.DMA,
        ],
    )
    out = run(y, idx)
    return out.reshape(BATCH, 1)
